# baseline (speedup 1.0000x reference)
import functools
import math

import jax
import jax.numpy as jnp
import numpy as np
from jax import lax
from jax.experimental import pallas as pl
from jax.experimental.pallas import tpu as pltpu
from jax.experimental.pallas import tpu_sc as plsc

_B = 4096
_S = 200
_D = 128
_PAD_IDX = 0

_NC = 2
_NS = 16
_NW = _NC * _NS
_ROWS_PER_W = _B // _NW
_ELEMS_PER_W = _ROWS_PER_W * _S
_C = 4
_GROUPS = _ROWS_PER_W // _C
_NBUF = 8
_LANES = 16
_GROUP_ELEMS = _C * _S
_VECS_PER_SCAN_BLOCK = 10
_SCAN_BLOCK = _VECS_PER_SCAN_BLOCK * _LANES
_BLOCKS_PER_GROUP = _GROUP_ELEMS // _SCAN_BLOCK
_U32_MAX = 4294967295


def _sinusoid_table() -> np.ndarray:
    half = _D // 2
    scale = math.log(10000.0) / (half - 1)
    freqs = np.exp(np.arange(half, dtype=np.float32) * np.float32(-scale))
    ang = np.arange(_S, dtype=np.float32)[:, None] * freqs[None, :]
    return np.concatenate([np.sin(ang), np.cos(ang)], axis=1).astype(np.float32)


_TABLE = _sinusoid_table()


@functools.partial(
    pl.kernel,
    mesh=plsc.VectorSubcoreMesh(core_axis_name="c", subcore_axis_name="s"),
    out_type=jax.ShapeDtypeStruct((_B * _S, _D), jnp.float32),
    scratch_types=[
        pltpu.VMEM((_C * _S, _D), jnp.float32),
        pltpu.VMEM((_ELEMS_PER_W,), jnp.int32),
        pltpu.SemaphoreType.DMA((_NBUF,)),
        pltpu.SemaphoreType.DMA,
    ],
)
def _pos_emb_sc(x_hbm, table_hbm, out_hbm, staging, x_v, sems, sem_x):
    wid = lax.axis_index("s") * _NC + lax.axis_index("c")
    base = wid * _ELEMS_PER_W

    x_load = pltpu.async_copy(x_hbm.at[pl.ds(base, _ELEMS_PER_W)], x_v, sem_x)
    stage_loads = [
        pltpu.async_copy(table_hbm, staging.at[pl.ds(c * _S, _S)], sems.at[c])
        for c in range(_C)
    ]
    for ld in stage_loads:
        ld.wait()

    def _issue(g, slot):
        pltpu.async_copy(
            staging,
            out_hbm.at[pl.ds(base + g * _GROUP_ELEMS, _GROUP_ELEMS)],
            sems.at[slot],
        )

    def _wait(slot):
        pltpu.make_async_copy(
            staging, out_hbm.at[pl.ds(base, _GROUP_ELEMS)], sems.at[slot]
        ).wait()

    def _scan_group(g):
        gl = g * _GROUP_ELEMS

        def _blk(b, _):
            l0 = pl.multiple_of(gl + b * _SCAN_BLOCK, _LANES)

            def _mn(i, acc):
                off = pl.multiple_of(l0 + i * _LANES, _LANES)
                v = plsc.bitcast(x_v[pl.ds(off, _LANES)], jnp.uint32)
                return jnp.minimum(acc, v)

            acc = lax.fori_loop(
                0, _VECS_PER_SCAN_BLOCK, _mn,
                jnp.full((_LANES,), _U32_MAX, jnp.uint32),
            )
            m = acc[0]
            for i in range(1, _LANES):
                m = jnp.minimum(m, acc[i])

            @pl.when(m == jnp.uint32(_PAD_IDX))
            def _():
                def _vec(i, _):
                    off = pl.multiple_of(l0 + i * _LANES, _LANES)
                    vals = x_v[pl.ds(off, _LANES)]
                    for lane in range(_LANES):

                        @pl.when(vals[lane] == _PAD_IDX)
                        def _():
                            pltpu.sync_copy(
                                staging.at[0], out_hbm.at[base + off + lane]
                            )

                    return 0

                lax.fori_loop(0, _VECS_PER_SCAN_BLOCK, _vec, 0)

            return 0

        lax.fori_loop(0, _BLOCKS_PER_GROUP, _blk, 0)

    for s in range(_NBUF):
        _issue(s, s)
    x_load.wait()

    def _main(g, _):
        slot = lax.rem(g, _NBUF)
        _wait(slot)
        _scan_group(g - _NBUF)
        _issue(g, slot)
        return 0

    lax.fori_loop(_NBUF, _GROUPS, _main, 0)

    def _tail(g, _):
        _wait(lax.rem(g, _NBUF))
        _scan_group(g)
        return 0

    lax.fori_loop(_GROUPS - _NBUF, _GROUPS, _tail, 0)


def kernel(x):
    x = x.astype(jnp.int32).reshape(_B * _S)
    out = _pos_emb_sc(x, jnp.asarray(_TABLE))
    return out.reshape(_B, _S, _D)


# speedup vs baseline: 1.0575x; 1.0575x over previous
import functools
import math

import jax
import jax.numpy as jnp
import numpy as np
from jax import lax
from jax.experimental import pallas as pl
from jax.experimental.pallas import tpu as pltpu
from jax.experimental.pallas import tpu_sc as plsc

_B = 4096
_S = 200
_D = 128
_PAD_IDX = 0

_NC = 2
_NS = 16
_NW = _NC * _NS
_ROWS_PER_W = _B // _NW
_ELEMS_PER_W = _ROWS_PER_W * _S
_C = 2
_GROUPS = _ROWS_PER_W // _C
_NBUF = 4
_LANES = 16
_GROUP_ELEMS = _C * _S
_VECS_PER_SCAN_BLOCK = 5
_SCAN_BLOCK = _VECS_PER_SCAN_BLOCK * _LANES
_BLOCKS_PER_GROUP = _GROUP_ELEMS // _SCAN_BLOCK
_U32_MAX = 4294967295


def _sinusoid_table() -> np.ndarray:
    half = _D // 2
    scale = math.log(10000.0) / (half - 1)
    freqs = np.exp(np.arange(half, dtype=np.float32) * np.float32(-scale))
    ang = np.arange(_S, dtype=np.float32)[:, None] * freqs[None, :]
    return np.concatenate([np.sin(ang), np.cos(ang)], axis=1).astype(np.float32)


_TABLE = _sinusoid_table()


@functools.partial(
    pl.kernel,
    mesh=plsc.VectorSubcoreMesh(core_axis_name="c", subcore_axis_name="s"),
    out_type=jax.ShapeDtypeStruct((_B * _S, _D), jnp.float32),
    scratch_types=[
        pltpu.VMEM((_C * _S, _D), jnp.float32),
        pltpu.VMEM((_ELEMS_PER_W,), jnp.int32),
        pltpu.SemaphoreType.DMA((_NBUF,)),
        pltpu.SemaphoreType.DMA,
    ],
)
def _pos_emb_sc(x_hbm, table_hbm, out_hbm, staging, x_v, sems, sem_x):
    wid = lax.axis_index("s") * _NC + lax.axis_index("c")
    base = wid * _ELEMS_PER_W

    x_load = pltpu.async_copy(x_hbm.at[pl.ds(base, _ELEMS_PER_W)], x_v, sem_x)
    stage_loads = [
        pltpu.async_copy(table_hbm, staging.at[pl.ds(c * _S, _S)], sems.at[c])
        for c in range(_C)
    ]
    for ld in stage_loads:
        ld.wait()

    def _issue(g, slot):
        pltpu.async_copy(
            staging,
            out_hbm.at[pl.ds(base + g * _GROUP_ELEMS, _GROUP_ELEMS)],
            sems.at[slot],
        )

    def _wait(slot):
        pltpu.make_async_copy(
            staging, out_hbm.at[pl.ds(base, _GROUP_ELEMS)], sems.at[slot]
        ).wait()

    def _scan_group(g):
        gl = g * _GROUP_ELEMS

        def _blk(b, _):
            l0 = pl.multiple_of(gl + b * _SCAN_BLOCK, _LANES)

            def _mn(i, acc):
                off = pl.multiple_of(l0 + i * _LANES, _LANES)
                v = plsc.bitcast(x_v[pl.ds(off, _LANES)], jnp.uint32)
                return jnp.minimum(acc, v)

            acc = lax.fori_loop(
                0, _VECS_PER_SCAN_BLOCK, _mn,
                jnp.full((_LANES,), _U32_MAX, jnp.uint32),
            )
            m = acc[0]
            for i in range(1, _LANES):
                m = jnp.minimum(m, acc[i])

            @pl.when(m == jnp.uint32(_PAD_IDX))
            def _():
                def _vec(i, _):
                    off = pl.multiple_of(l0 + i * _LANES, _LANES)
                    vals = x_v[pl.ds(off, _LANES)]
                    for lane in range(_LANES):

                        @pl.when(vals[lane] == _PAD_IDX)
                        def _():
                            pltpu.sync_copy(
                                staging.at[0], out_hbm.at[base + off + lane]
                            )

                    return 0

                lax.fori_loop(0, _VECS_PER_SCAN_BLOCK, _vec, 0)

            return 0

        lax.fori_loop(0, _BLOCKS_PER_GROUP, _blk, 0)

    for s in range(_NBUF):
        _issue(s, s)
    x_load.wait()

    def _main(g, _):
        slot = lax.rem(g, _NBUF)
        _wait(slot)
        _scan_group(g - _NBUF)
        _issue(g, slot)
        return 0

    lax.fori_loop(_NBUF, _GROUPS, _main, 0)

    def _tail(g, _):
        _wait(lax.rem(g, _NBUF))
        _scan_group(g)
        return 0

    lax.fori_loop(_GROUPS - _NBUF, _GROUPS, _tail, 0)


def kernel(x):
    x = x.astype(jnp.int32).reshape(_B * _S)
    out = _pos_emb_sc(x, jnp.asarray(_TABLE))
    return out.reshape(_B, _S, _D)


# speedup vs baseline: 1.0576x; 1.0001x over previous
import functools
import math

import jax
import jax.numpy as jnp
import numpy as np
from jax import lax
from jax.experimental import pallas as pl
from jax.experimental.pallas import tpu as pltpu
from jax.experimental.pallas import tpu_sc as plsc

_B = 4096
_S = 200
_D = 128
_PAD_IDX = 0

_NC = 2
_NS = 16
_NW = _NC * _NS
_ROWS_PER_W = _B // _NW
_ELEMS_PER_W = _ROWS_PER_W * _S
_C = 2
_GROUPS = _ROWS_PER_W // _C
_NBUF = 8
_LANES = 16
_GROUP_ELEMS = _C * _S
_VECS_PER_SCAN_BLOCK = 5
_SCAN_BLOCK = _VECS_PER_SCAN_BLOCK * _LANES
_BLOCKS_PER_GROUP = _GROUP_ELEMS // _SCAN_BLOCK
_U32_MAX = 4294967295


def _sinusoid_table() -> np.ndarray:
    half = _D // 2
    scale = math.log(10000.0) / (half - 1)
    freqs = np.exp(np.arange(half, dtype=np.float32) * np.float32(-scale))
    ang = np.arange(_S, dtype=np.float32)[:, None] * freqs[None, :]
    return np.concatenate([np.sin(ang), np.cos(ang)], axis=1).astype(np.float32)


_TABLE = _sinusoid_table()


@functools.partial(
    pl.kernel,
    mesh=plsc.VectorSubcoreMesh(core_axis_name="c", subcore_axis_name="s"),
    out_type=jax.ShapeDtypeStruct((_B * _S, _D), jnp.float32),
    scratch_types=[
        pltpu.VMEM((_C * _S, _D), jnp.float32),
        pltpu.VMEM((_ELEMS_PER_W,), jnp.int32),
        pltpu.SemaphoreType.DMA((_NBUF,)),
        pltpu.SemaphoreType.DMA,
    ],
)
def _pos_emb_sc(x_hbm, table_hbm, out_hbm, staging, x_v, sems, sem_x):
    wid = lax.axis_index("s") * _NC + lax.axis_index("c")
    base = wid * _ELEMS_PER_W

    x_load = pltpu.async_copy(x_hbm.at[pl.ds(base, _ELEMS_PER_W)], x_v, sem_x)
    stage_loads = [
        pltpu.async_copy(table_hbm, staging.at[pl.ds(c * _S, _S)], sems.at[c])
        for c in range(_C)
    ]
    for ld in stage_loads:
        ld.wait()

    def _issue(g, slot):
        pltpu.async_copy(
            staging,
            out_hbm.at[pl.ds(base + g * _GROUP_ELEMS, _GROUP_ELEMS)],
            sems.at[slot],
        )

    def _wait(slot):
        pltpu.make_async_copy(
            staging, out_hbm.at[pl.ds(base, _GROUP_ELEMS)], sems.at[slot]
        ).wait()

    def _scan_group(g):
        gl = g * _GROUP_ELEMS

        def _blk(b, _):
            l0 = pl.multiple_of(gl + b * _SCAN_BLOCK, _LANES)

            def _mn(i, acc):
                off = pl.multiple_of(l0 + i * _LANES, _LANES)
                v = plsc.bitcast(x_v[pl.ds(off, _LANES)], jnp.uint32)
                return jnp.minimum(acc, v)

            acc = lax.fori_loop(
                0, _VECS_PER_SCAN_BLOCK, _mn,
                jnp.full((_LANES,), _U32_MAX, jnp.uint32),
            )
            m = acc[0]
            for i in range(1, _LANES):
                m = jnp.minimum(m, acc[i])

            @pl.when(m == jnp.uint32(_PAD_IDX))
            def _():
                def _vec(i, _):
                    off = pl.multiple_of(l0 + i * _LANES, _LANES)
                    vals = x_v[pl.ds(off, _LANES)]
                    for lane in range(_LANES):

                        @pl.when(vals[lane] == _PAD_IDX)
                        def _():
                            pltpu.sync_copy(
                                staging.at[0], out_hbm.at[base + off + lane]
                            )

                    return 0

                lax.fori_loop(0, _VECS_PER_SCAN_BLOCK, _vec, 0)

            return 0

        lax.fori_loop(0, _BLOCKS_PER_GROUP, _blk, 0)

    for s in range(_NBUF):
        _issue(s, s)
    x_load.wait()

    def _main(g, _):
        slot = lax.rem(g, _NBUF)
        _wait(slot)
        _scan_group(g - _NBUF)
        _issue(g, slot)
        return 0

    lax.fori_loop(_NBUF, _GROUPS, _main, 0)

    def _tail(g, _):
        _wait(lax.rem(g, _NBUF))
        _scan_group(g)
        return 0

    lax.fori_loop(_GROUPS - _NBUF, _GROUPS, _tail, 0)


def kernel(x):
    x = x.astype(jnp.int32).reshape(_B * _S)
    out = _pos_emb_sc(x, jnp.asarray(_TABLE))
    return out.reshape(_B, _S, _D)
